# baseline (speedup 1.0000x reference)
import functools

import jax
import jax.numpy as jnp
from jax import lax
from jax.experimental import pallas as pl
from jax.experimental.pallas import tpu as pltpu
from jax.experimental.pallas import tpu_sc as plsc

N_BOOKS = 100000
B = 4096
H = 200
D_ITEM = 64
N_SHELVES = 64
D_AUTH = 16
D_YEAR = 16
D_TS = 32

NC = 2
NS = 16
NW = NC * NS
ROWS_PER_W = B // NW
HA = 112
HB = H - HA


def _sc_body(item_hbm, shelf_hbm, auth_hbm, year_hbm, ts_hbm,
             xh_hbm, rat_hbm, tb_hbm, ta_hbm, ty_hbm, tt_hbm,
             hist_out, wsum_out, titem_out, tshelf_out, tauth_out,
             tyear_out, tts_out,
             idx128, buf16a, buf16b, buf32,
             idx_all, rat_all, ra0, rb0, ra1, rb1, ra2, rb2, ra3, rb3,
             histbuf, wsbuf,
             sem, s0, s1, s2, s3):
    wid = lax.axis_index("s") * NC + lax.axis_index("c")
    base = wid * ROWS_PER_W

    pltpu.sync_copy(tb_hbm.at[pl.ds(base, ROWS_PER_W)], idx128)
    pltpu.async_copy(item_hbm.at[idx128], histbuf, sem).wait()
    pltpu.sync_copy(histbuf, titem_out.at[pl.ds(base, ROWS_PER_W)])
    pltpu.async_copy(shelf_hbm.at[idx128], histbuf, sem).wait()
    pltpu.sync_copy(histbuf, tshelf_out.at[pl.ds(base, ROWS_PER_W)])

    pltpu.sync_copy(ta_hbm.at[pl.ds(base, ROWS_PER_W)], idx128)
    pltpu.async_copy(auth_hbm.at[idx128], buf16a, sem).wait()
    pltpu.sync_copy(buf16a, tauth_out.at[pl.ds(base, ROWS_PER_W)])

    pltpu.sync_copy(ty_hbm.at[pl.ds(base, ROWS_PER_W)], idx128)
    pltpu.async_copy(year_hbm.at[idx128], buf16b, sem).wait()
    pltpu.sync_copy(buf16b, tyear_out.at[pl.ds(base, ROWS_PER_W)])

    pltpu.sync_copy(tt_hbm.at[pl.ds(base, ROWS_PER_W)], idx128)
    pltpu.async_copy(ts_hbm.at[idx128], buf32, sem).wait()
    pltpu.sync_copy(buf32, tts_out.at[pl.ds(base, ROWS_PER_W)])

    pltpu.sync_copy(xh_hbm.at[pl.ds(base * H, ROWS_PER_W * H)], idx_all)
    pltpu.sync_copy(rat_hbm.at[pl.ds(base * H, ROWS_PER_W * H)], rat_all)

    lanes = lax.iota(jnp.int32, 16)

    def remap_step(c, _):
        off = c * 16
        iv = idx_all[pl.ds(off, 16)]
        rv = rat_all[pl.ds(off, 16)]
        valid = iv != N_BOOKS
        spread = jnp.bitwise_and(lanes + off + wid * 811, 65535)
        idx_all[pl.ds(off, 16)] = jnp.where(valid, iv, spread)
        rat_all[pl.ds(off, 16)] = jnp.where(valid, rv, jnp.float32(0.0))
        return _

    lax.fori_loop(0, ROWS_PER_W * H // 16, remap_step, 0)

    ras = (ra0, ra1, ra2, ra3)
    rbs = (rb0, rb1, rb2, rb3)
    sems = (s0, s1, s2, s3)
    DEPTH = 4

    def start_gather(r, rows_a, rows_b, dsem):
        off = r * H
        pltpu.async_copy(item_hbm.at[idx_all.at[pl.ds(off, HA)]],
                         rows_a, dsem)
        pltpu.async_copy(item_hbm.at[idx_all.at[pl.ds(off + HA, HB)]],
                         rows_b, dsem)

    def drain(rows_a, rows_b, dsem):
        pltpu.make_async_copy(item_hbm.at[idx_all.at[pl.ds(0, HA)]],
                              rows_a, dsem).wait()
        pltpu.make_async_copy(item_hbm.at[idx_all.at[pl.ds(0, HB)]],
                              rows_b, dsem).wait()

    def compute_row(r, rows_a, rows_b):
        def make_chunk_step(rows_ref, hoff):
            def chunk_step(k, carry):
                a0, a1, a2, a3, wsv = carry
                h0 = k * 16
                wv = rat_all[pl.ds(r * H + hoff + h0, 16)]
                wsv = wsv + jnp.abs(wv)
                for j in range(2):
                    w = wv[j]
                    h = h0 + j
                    a0 = a0 + rows_ref[h, pl.ds(0, 16)] * w
                    a1 = a1 + rows_ref[h, pl.ds(16, 16)] * w
                    a2 = a2 + rows_ref[h, pl.ds(32, 16)] * w
                    a3 = a3 + rows_ref[h, pl.ds(48, 16)] * w
                return a0, a1, a2, a3, wsv
            return chunk_step

        z = jnp.zeros((16,), jnp.float32)
        carry = (z, z, z, z, z)
        carry = lax.fori_loop(0, HA // 16, make_chunk_step(rows_a, 0), carry)
        a0, a1, a2, a3, wsv = lax.fori_loop(
            0, (HB - 8) // 16, make_chunk_step(rows_b, HA), carry)
        wv_t = rat_all[pl.ds(r * H + H - 16, 16)]
        wsv = wsv + jnp.abs(jnp.where(lanes >= 8, wv_t, jnp.float32(0.0)))
        for j in range(8, 16):
            w = wv_t[j]
            h = (H - 16 - HA) + j
            a0 = a0 + rows_b[h, pl.ds(0, 16)] * w
            a1 = a1 + rows_b[h, pl.ds(16, 16)] * w
            a2 = a2 + rows_b[h, pl.ds(32, 16)] * w
            a3 = a3 + rows_b[h, pl.ds(48, 16)] * w
        histbuf[r, pl.ds(0, 16)] = a0
        histbuf[r, pl.ds(16, 16)] = a1
        histbuf[r, pl.ds(32, 16)] = a2
        histbuf[r, pl.ds(48, 16)] = a3
        wsbuf[r, pl.ds(0, 16)] = wsv

    for s in range(DEPTH):
        start_gather(s, ras[s], rbs[s], sems[s])

    def ring_step(q, _):
        for s in range(DEPTH):
            r = DEPTH * q + s
            drain(ras[s], rbs[s], sems[s])
            compute_row(r, ras[s], rbs[s])
            rn = jnp.minimum(r + DEPTH, ROWS_PER_W - 1)
            start_gather(rn, ras[s], rbs[s], sems[s])
        return _

    lax.fori_loop(0, ROWS_PER_W // DEPTH, ring_step, 0)
    for s in range(DEPTH):
        drain(ras[s], rbs[s], sems[s])
    pltpu.sync_copy(histbuf, hist_out.at[pl.ds(base, ROWS_PER_W)])
    pltpu.sync_copy(wsbuf, wsum_out.at[pl.ds(base, ROWS_PER_W)])


@jax.jit
def _sc_gather(item_table, shelf_matrix, author_table, year_table, ts_table,
               x_history, ratings, tb_idx, ta_idx, ty_idx, tt_idx):
    mesh = plsc.VectorSubcoreMesh(core_axis_name="c", subcore_axis_name="s")
    f = pl.kernel(
        _sc_body,
        out_type=[
            jax.ShapeDtypeStruct((B, D_ITEM), jnp.float32),
            jax.ShapeDtypeStruct((B, 16), jnp.float32),
            jax.ShapeDtypeStruct((B, D_ITEM), jnp.float32),
            jax.ShapeDtypeStruct((B, N_SHELVES), jnp.float32),
            jax.ShapeDtypeStruct((B, D_AUTH), jnp.float32),
            jax.ShapeDtypeStruct((B, D_YEAR), jnp.float32),
            jax.ShapeDtypeStruct((B, D_TS), jnp.float32),
        ],
        mesh=mesh,
        compiler_params=pltpu.CompilerParams(use_tc_tiling_on_sc=False),
        scratch_types=[
            pltpu.VMEM((ROWS_PER_W,), jnp.int32),
            pltpu.VMEM((ROWS_PER_W, D_AUTH), jnp.float32),
            pltpu.VMEM((ROWS_PER_W, D_YEAR), jnp.float32),
            pltpu.VMEM((ROWS_PER_W, D_TS), jnp.float32),
            pltpu.VMEM((ROWS_PER_W * H,), jnp.int32),
            pltpu.VMEM((ROWS_PER_W * H,), jnp.float32),
            pltpu.VMEM((HA, D_ITEM), jnp.float32),
            pltpu.VMEM((HB, D_ITEM), jnp.float32),
            pltpu.VMEM((HA, D_ITEM), jnp.float32),
            pltpu.VMEM((HB, D_ITEM), jnp.float32),
            pltpu.VMEM((HA, D_ITEM), jnp.float32),
            pltpu.VMEM((HB, D_ITEM), jnp.float32),
            pltpu.VMEM((HA, D_ITEM), jnp.float32),
            pltpu.VMEM((HB, D_ITEM), jnp.float32),
            pltpu.VMEM((ROWS_PER_W, D_ITEM), jnp.float32),
            pltpu.VMEM((ROWS_PER_W, 16), jnp.float32),
            pltpu.SemaphoreType.DMA,
            pltpu.SemaphoreType.DMA,
            pltpu.SemaphoreType.DMA,
            pltpu.SemaphoreType.DMA,
            pltpu.SemaphoreType.DMA,
        ],
    )
    return f(item_table, shelf_matrix, author_table, year_table, ts_table,
             x_history, ratings, tb_idx, ta_idx, ty_idx, tt_idx)


def _tc_body(hist_ref, wsum_ref, xg_ref, tg_ref, tts_ref, titem_ref,
             tshelf_ref, tauth_ref, tyear_ref,
             wug_ref, bug_ref, wts_ref, bts_ref, wig_ref, big_ref,
             wsh_ref, bsh_ref, wit_ref, bit_ref, wau_ref, bau_ref,
             wyr_ref, byr_ref, out_ref):
    wsum = jnp.maximum(jnp.sum(wsum_ref[...], axis=1, keepdims=True), 1e-6)
    hist = hist_ref[...] / wsum
    g = jnp.tanh(jnp.dot(xg_ref[...], wug_ref[...],
                         preferred_element_type=jnp.float32) + bug_ref[...])
    ts_e = jnp.tanh(jnp.dot(tts_ref[...], wts_ref[...],
                            preferred_element_type=jnp.float32) + bts_ref[...])
    ig = jnp.tanh(jnp.dot(tg_ref[...], wig_ref[...],
                          preferred_element_type=jnp.float32) + big_ref[...])
    sh = jnp.tanh(jnp.dot(tshelf_ref[...], wsh_ref[...],
                          preferred_element_type=jnp.float32) + bsh_ref[...])
    it = jnp.tanh(jnp.dot(titem_ref[...], wit_ref[...],
                          preferred_element_type=jnp.float32) + bit_ref[...])
    au = jnp.tanh(jnp.dot(tauth_ref[...], wau_ref[...],
                          preferred_element_type=jnp.float32) + bau_ref[...])
    yr = jnp.tanh(jnp.dot(tyear_ref[...], wyr_ref[...],
                          preferred_element_type=jnp.float32) + byr_ref[...])
    dot = (jnp.sum(hist[:, :32] * ig, axis=1, keepdims=True)
           + jnp.sum(hist[:, 32:] * sh, axis=1, keepdims=True)
           + jnp.sum(g * it, axis=1, keepdims=True)
           + jnp.sum(ts_e[:, :16] * au, axis=1, keepdims=True)
           + jnp.sum(ts_e[:, 16:] * yr, axis=1, keepdims=True))
    out_ref[...] = dot


def kernel(X_genre, X_history, X_history_ratings, timestamps, target_genre,
           target_year, target_book_idx, target_author_idx, item_table,
           author_table, year_table, ts_table, shelf_matrix, W_item, b_item,
           W_auth, b_auth, W_shelf, b_shelf, W_ig, b_ig, W_yr, b_yr, W_ug,
           b_ug, W_ts, b_ts):
    xh = X_history.astype(jnp.int32).reshape(B * H)
    rat = X_history_ratings.reshape(B * H)
    tb = target_book_idx.astype(jnp.int32)
    ta = target_author_idx.astype(jnp.int32)
    ty = target_year.astype(jnp.int32)
    tt = timestamps.astype(jnp.int32)

    (hist_raw, wsum, titem, tshelf, tauth, tyear, tts) = _sc_gather(
        item_table, shelf_matrix, author_table, year_table, ts_table,
        xh, rat, tb, ta, ty, tt)

    nblk = 8
    bs = B // nblk
    rep = lambda shape: pl.BlockSpec(shape, lambda i: (0,) * len(shape))
    blk = lambda d: pl.BlockSpec((bs, d), lambda i: (i, 0))
    out = pl.pallas_call(
        _tc_body,
        grid=(nblk,),
        in_specs=[
            blk(D_ITEM), blk(16), blk(X_genre.shape[1]), blk(target_genre.shape[1]),
            blk(D_TS), blk(D_ITEM), blk(N_SHELVES), blk(D_AUTH), blk(D_YEAR),
            rep(W_ug.shape), rep((1, b_ug.shape[0])),
            rep(W_ts.shape), rep((1, b_ts.shape[0])),
            rep(W_ig.shape), rep((1, b_ig.shape[0])),
            rep(W_shelf.shape), rep((1, b_shelf.shape[0])),
            rep(W_item.shape), rep((1, b_item.shape[0])),
            rep(W_auth.shape), rep((1, b_auth.shape[0])),
            rep(W_yr.shape), rep((1, b_yr.shape[0])),
        ],
        out_specs=blk(1),
        out_shape=jax.ShapeDtypeStruct((B, 1), jnp.float32),
    )(hist_raw, wsum, X_genre, target_genre, tts, titem,
      tshelf, tauth, tyear,
      W_ug, b_ug.reshape(1, -1), W_ts, b_ts.reshape(1, -1),
      W_ig, b_ig.reshape(1, -1), W_shelf, b_shelf.reshape(1, -1),
      W_item, b_item.reshape(1, -1), W_auth, b_auth.reshape(1, -1),
      W_yr, b_yr.reshape(1, -1))
    return out.reshape(B)


# speedup vs baseline: 1.0460x; 1.0460x over previous
import functools

import jax
import jax.numpy as jnp
from jax import lax
from jax.experimental import pallas as pl
from jax.experimental.pallas import tpu as pltpu
from jax.experimental.pallas import tpu_sc as plsc

N_BOOKS = 100000
B = 4096
H = 200
D_ITEM = 64
N_SHELVES = 64
D_AUTH = 16
D_YEAR = 16
D_TS = 32

NC = 2
NS = 16
NW = NC * NS
ROWS_PER_W = B // NW
HA = 112
HB = H - HA


def _sc_body(item_hbm, shelf_hbm, auth_hbm, year_hbm, ts_hbm,
             xh_hbm, rat_hbm, tb_hbm, ta_hbm, ty_hbm, tt_hbm,
             hist_out, wsum_out, titem_out, tshelf_out, tauth_out,
             tyear_out, tts_out,
             idx128, buf16a, buf16b, buf32,
             idx_all, rat_all, ra0, rb0, ra1, rb1, ra2, rb2, ra3, rb3,
             histbuf, wsbuf,
             sem, s0, s1, s2, s3):
    wid = lax.axis_index("s") * NC + lax.axis_index("c")
    base = wid * ROWS_PER_W

    pltpu.sync_copy(tb_hbm.at[pl.ds(base, ROWS_PER_W)], idx128)
    pltpu.async_copy(item_hbm.at[idx128], histbuf, sem).wait()
    pltpu.sync_copy(histbuf, titem_out.at[pl.ds(base, ROWS_PER_W)])
    pltpu.async_copy(shelf_hbm.at[idx128], histbuf, sem).wait()
    pltpu.sync_copy(histbuf, tshelf_out.at[pl.ds(base, ROWS_PER_W)])

    pltpu.sync_copy(ta_hbm.at[pl.ds(base, ROWS_PER_W)], idx128)
    pltpu.async_copy(auth_hbm.at[idx128], buf16a, sem).wait()
    pltpu.sync_copy(buf16a, tauth_out.at[pl.ds(base, ROWS_PER_W)])

    pltpu.sync_copy(ty_hbm.at[pl.ds(base, ROWS_PER_W)], idx128)
    pltpu.async_copy(year_hbm.at[idx128], buf16b, sem).wait()
    pltpu.sync_copy(buf16b, tyear_out.at[pl.ds(base, ROWS_PER_W)])

    pltpu.sync_copy(tt_hbm.at[pl.ds(base, ROWS_PER_W)], idx128)
    pltpu.async_copy(ts_hbm.at[idx128], buf32, sem).wait()
    pltpu.sync_copy(buf32, tts_out.at[pl.ds(base, ROWS_PER_W)])

    pltpu.sync_copy(xh_hbm.at[pl.ds(base * H, ROWS_PER_W * H)], idx_all)
    pltpu.sync_copy(rat_hbm.at[pl.ds(base * H, ROWS_PER_W * H)], rat_all)

    lanes = lax.iota(jnp.int32, 16)

    def remap_step(c, _):
        off = c * 16
        iv = idx_all[pl.ds(off, 16)]
        rv = rat_all[pl.ds(off, 16)]
        valid = iv != N_BOOKS
        spread = jnp.bitwise_and(lanes + off + wid * 811, 65535)
        idx_all[pl.ds(off, 16)] = jnp.where(valid, iv, spread)
        rat_all[pl.ds(off, 16)] = jnp.where(valid, rv, jnp.float32(0.0))
        return _

    lax.fori_loop(0, ROWS_PER_W * H // 16, remap_step, 0)

    ras = (ra0, ra1, ra2, ra3)
    rbs = (rb0, rb1, rb2, rb3)
    sems = (s0, s1, s2, s3)
    DEPTH = 4

    def start_gather(r, rows_a, rows_b, dsem):
        off = r * H
        pltpu.async_copy(item_hbm.at[idx_all.at[pl.ds(off, HA)]],
                         rows_a, dsem)
        pltpu.async_copy(item_hbm.at[idx_all.at[pl.ds(off + HA, HB)]],
                         rows_b, dsem)

    def drain(rows_a, rows_b, dsem):
        pltpu.make_async_copy(item_hbm.at[idx_all.at[pl.ds(0, HA)]],
                              rows_a, dsem).wait()
        pltpu.make_async_copy(item_hbm.at[idx_all.at[pl.ds(0, HB)]],
                              rows_b, dsem).wait()

    def compute_row(r, rows_a, rows_b):
        def make_chunk_step(rows_ref, hoff):
            def chunk_step(k, carry):
                a0, a1, a2, a3, wsv = carry
                h0 = k * 16
                wv = rat_all[pl.ds(r * H + hoff + h0, 16)]
                wsv = wsv + jnp.abs(wv)
                for j in range(16):
                    w = wv[j]
                    h = h0 + j
                    a0 = a0 + rows_ref[h, pl.ds(0, 16)] * w
                    a1 = a1 + rows_ref[h, pl.ds(16, 16)] * w
                    a2 = a2 + rows_ref[h, pl.ds(32, 16)] * w
                    a3 = a3 + rows_ref[h, pl.ds(48, 16)] * w
                return a0, a1, a2, a3, wsv
            return chunk_step

        z = jnp.zeros((16,), jnp.float32)
        carry = (z, z, z, z, z)
        carry = lax.fori_loop(0, HA // 16, make_chunk_step(rows_a, 0), carry)
        a0, a1, a2, a3, wsv = lax.fori_loop(
            0, (HB - 8) // 16, make_chunk_step(rows_b, HA), carry)
        wv_t = rat_all[pl.ds(r * H + H - 16, 16)]
        wsv = wsv + jnp.abs(jnp.where(lanes >= 8, wv_t, jnp.float32(0.0)))
        for j in range(8, 16):
            w = wv_t[j]
            h = (H - 16 - HA) + j
            a0 = a0 + rows_b[h, pl.ds(0, 16)] * w
            a1 = a1 + rows_b[h, pl.ds(16, 16)] * w
            a2 = a2 + rows_b[h, pl.ds(32, 16)] * w
            a3 = a3 + rows_b[h, pl.ds(48, 16)] * w
        histbuf[r, pl.ds(0, 16)] = a0
        histbuf[r, pl.ds(16, 16)] = a1
        histbuf[r, pl.ds(32, 16)] = a2
        histbuf[r, pl.ds(48, 16)] = a3
        wsbuf[r, pl.ds(0, 16)] = wsv

    for s in range(DEPTH):
        start_gather(s, ras[s], rbs[s], sems[s])

    def ring_step(q, _):
        for s in range(DEPTH):
            r = DEPTH * q + s
            drain(ras[s], rbs[s], sems[s])
            compute_row(r, ras[s], rbs[s])
            rn = jnp.minimum(r + DEPTH, ROWS_PER_W - 1)
            start_gather(rn, ras[s], rbs[s], sems[s])
        return _

    lax.fori_loop(0, ROWS_PER_W // DEPTH, ring_step, 0)
    for s in range(DEPTH):
        drain(ras[s], rbs[s], sems[s])
    pltpu.sync_copy(histbuf, hist_out.at[pl.ds(base, ROWS_PER_W)])
    pltpu.sync_copy(wsbuf, wsum_out.at[pl.ds(base, ROWS_PER_W)])


@jax.jit
def _sc_gather(item_table, shelf_matrix, author_table, year_table, ts_table,
               x_history, ratings, tb_idx, ta_idx, ty_idx, tt_idx):
    mesh = plsc.VectorSubcoreMesh(core_axis_name="c", subcore_axis_name="s")
    f = pl.kernel(
        _sc_body,
        out_type=[
            jax.ShapeDtypeStruct((B, D_ITEM), jnp.float32),
            jax.ShapeDtypeStruct((B, 16), jnp.float32),
            jax.ShapeDtypeStruct((B, D_ITEM), jnp.float32),
            jax.ShapeDtypeStruct((B, N_SHELVES), jnp.float32),
            jax.ShapeDtypeStruct((B, D_AUTH), jnp.float32),
            jax.ShapeDtypeStruct((B, D_YEAR), jnp.float32),
            jax.ShapeDtypeStruct((B, D_TS), jnp.float32),
        ],
        mesh=mesh,
        compiler_params=pltpu.CompilerParams(use_tc_tiling_on_sc=False),
        scratch_types=[
            pltpu.VMEM((ROWS_PER_W,), jnp.int32),
            pltpu.VMEM((ROWS_PER_W, D_AUTH), jnp.float32),
            pltpu.VMEM((ROWS_PER_W, D_YEAR), jnp.float32),
            pltpu.VMEM((ROWS_PER_W, D_TS), jnp.float32),
            pltpu.VMEM((ROWS_PER_W * H,), jnp.int32),
            pltpu.VMEM((ROWS_PER_W * H,), jnp.float32),
            pltpu.VMEM((HA, D_ITEM), jnp.float32),
            pltpu.VMEM((HB, D_ITEM), jnp.float32),
            pltpu.VMEM((HA, D_ITEM), jnp.float32),
            pltpu.VMEM((HB, D_ITEM), jnp.float32),
            pltpu.VMEM((HA, D_ITEM), jnp.float32),
            pltpu.VMEM((HB, D_ITEM), jnp.float32),
            pltpu.VMEM((HA, D_ITEM), jnp.float32),
            pltpu.VMEM((HB, D_ITEM), jnp.float32),
            pltpu.VMEM((ROWS_PER_W, D_ITEM), jnp.float32),
            pltpu.VMEM((ROWS_PER_W, 16), jnp.float32),
            pltpu.SemaphoreType.DMA,
            pltpu.SemaphoreType.DMA,
            pltpu.SemaphoreType.DMA,
            pltpu.SemaphoreType.DMA,
            pltpu.SemaphoreType.DMA,
        ],
    )
    return f(item_table, shelf_matrix, author_table, year_table, ts_table,
             x_history, ratings, tb_idx, ta_idx, ty_idx, tt_idx)


def _tc_body(hist_ref, wsum_ref, xg_ref, tg_ref, tts_ref, titem_ref,
             tshelf_ref, tauth_ref, tyear_ref,
             wug_ref, bug_ref, wts_ref, bts_ref, wig_ref, big_ref,
             wsh_ref, bsh_ref, wit_ref, bit_ref, wau_ref, bau_ref,
             wyr_ref, byr_ref, out_ref):
    wsum = jnp.maximum(jnp.sum(wsum_ref[...], axis=1, keepdims=True), 1e-6)
    hist = hist_ref[...] / wsum
    g = jnp.tanh(jnp.dot(xg_ref[...], wug_ref[...],
                         preferred_element_type=jnp.float32) + bug_ref[...])
    ts_e = jnp.tanh(jnp.dot(tts_ref[...], wts_ref[...],
                            preferred_element_type=jnp.float32) + bts_ref[...])
    ig = jnp.tanh(jnp.dot(tg_ref[...], wig_ref[...],
                          preferred_element_type=jnp.float32) + big_ref[...])
    sh = jnp.tanh(jnp.dot(tshelf_ref[...], wsh_ref[...],
                          preferred_element_type=jnp.float32) + bsh_ref[...])
    it = jnp.tanh(jnp.dot(titem_ref[...], wit_ref[...],
                          preferred_element_type=jnp.float32) + bit_ref[...])
    au = jnp.tanh(jnp.dot(tauth_ref[...], wau_ref[...],
                          preferred_element_type=jnp.float32) + bau_ref[...])
    yr = jnp.tanh(jnp.dot(tyear_ref[...], wyr_ref[...],
                          preferred_element_type=jnp.float32) + byr_ref[...])
    dot = (jnp.sum(hist[:, :32] * ig, axis=1, keepdims=True)
           + jnp.sum(hist[:, 32:] * sh, axis=1, keepdims=True)
           + jnp.sum(g * it, axis=1, keepdims=True)
           + jnp.sum(ts_e[:, :16] * au, axis=1, keepdims=True)
           + jnp.sum(ts_e[:, 16:] * yr, axis=1, keepdims=True))
    out_ref[...] = dot


def kernel(X_genre, X_history, X_history_ratings, timestamps, target_genre,
           target_year, target_book_idx, target_author_idx, item_table,
           author_table, year_table, ts_table, shelf_matrix, W_item, b_item,
           W_auth, b_auth, W_shelf, b_shelf, W_ig, b_ig, W_yr, b_yr, W_ug,
           b_ug, W_ts, b_ts):
    xh = X_history.astype(jnp.int32).reshape(B * H)
    rat = X_history_ratings.reshape(B * H)
    tb = target_book_idx.astype(jnp.int32)
    ta = target_author_idx.astype(jnp.int32)
    ty = target_year.astype(jnp.int32)
    tt = timestamps.astype(jnp.int32)

    (hist_raw, wsum, titem, tshelf, tauth, tyear, tts) = _sc_gather(
        item_table, shelf_matrix, author_table, year_table, ts_table,
        xh, rat, tb, ta, ty, tt)

    nblk = 8
    bs = B // nblk
    rep = lambda shape: pl.BlockSpec(shape, lambda i: (0,) * len(shape))
    blk = lambda d: pl.BlockSpec((bs, d), lambda i: (i, 0))
    out = pl.pallas_call(
        _tc_body,
        grid=(nblk,),
        in_specs=[
            blk(D_ITEM), blk(16), blk(X_genre.shape[1]), blk(target_genre.shape[1]),
            blk(D_TS), blk(D_ITEM), blk(N_SHELVES), blk(D_AUTH), blk(D_YEAR),
            rep(W_ug.shape), rep((1, b_ug.shape[0])),
            rep(W_ts.shape), rep((1, b_ts.shape[0])),
            rep(W_ig.shape), rep((1, b_ig.shape[0])),
            rep(W_shelf.shape), rep((1, b_shelf.shape[0])),
            rep(W_item.shape), rep((1, b_item.shape[0])),
            rep(W_auth.shape), rep((1, b_auth.shape[0])),
            rep(W_yr.shape), rep((1, b_yr.shape[0])),
        ],
        out_specs=blk(1),
        out_shape=jax.ShapeDtypeStruct((B, 1), jnp.float32),
    )(hist_raw, wsum, X_genre, target_genre, tts, titem,
      tshelf, tauth, tyear,
      W_ug, b_ug.reshape(1, -1), W_ts, b_ts.reshape(1, -1),
      W_ig, b_ig.reshape(1, -1), W_shelf, b_shelf.reshape(1, -1),
      W_item, b_item.reshape(1, -1), W_auth, b_auth.reshape(1, -1),
      W_yr, b_yr.reshape(1, -1))
    return out.reshape(B)
